# baseline (speedup 1.0000x reference)
import jax
import jax.numpy as jnp
from jax import lax
from jax.experimental import pallas as pl
from jax.experimental.pallas import tpu as pltpu
from jax.experimental.pallas import tpu_sc as plsc

VOCAB = 100000
D = 128
B = 16384
NEG = 5

NC = 2
NS = 16
NW = NC * NS
SAMP_W = B // NW
C = 32
NCH = SAMP_W // C
GROUPS = C // 16

_TWO_PI = 6.283185307179586
_PI = 3.141592653589793
_HALF_PI = 1.5707963267948966
_S3 = -0.16666666666666666
_S5 = 0.008333333333333333
_S7 = -0.0001984126984126984
_S9 = 2.7557319223985893e-06


def _sin16(x):
  r = lax.rem(x, jnp.float32(_TWO_PI))
  r = jnp.where(r > _PI, r - _TWO_PI, r)
  r = jnp.where(r < -_PI, r + _TWO_PI, r)
  r = jnp.where(r > _HALF_PI, _PI - r, r)
  r = jnp.where(r < -_HALF_PI, -_PI - r, r)
  r2 = r * r
  p = jnp.float32(_S9)
  p = p * r2 + _S7
  p = p * r2 + _S5
  p = p * r2 + _S3
  p = p * r2 + 1.0
  return p * r


def _sc_body(u_hbm, v_hbm, pu_hbm, pv_hbm, nf_hbm, t_hbm, f_hbm,
             ps_hbm, ns_hbm,
             idxu, idxv, idxn, tbuf, fsplat, pos_w, neg_w,
             ub0, vb0, nb0, ub1, vb1, nb1,
             psem, g0, g1, wsem):
  c = lax.axis_index("c")
  s = lax.axis_index("s")
  wid = s * NC + c
  base = wid * SAMP_W

  ubufs = (ub0, ub1)
  vbufs = (vb0, vb1)
  nbufs = (nb0, nb1)
  gsem = (g0, g1)

  hs = [
      pltpu.async_copy(pu_hbm.at[pl.ds(base, SAMP_W)], idxu, psem),
      pltpu.async_copy(pv_hbm.at[pl.ds(base, SAMP_W)], idxv, psem),
      pltpu.async_copy(t_hbm.at[pl.ds(base, SAMP_W)], tbuf, psem),
      pltpu.async_copy(f_hbm, fsplat, psem),
  ]
  for k in range(NEG):
    hs.append(pltpu.async_copy(nf_hbm.at[pl.ds(k * B + base, SAMP_W)],
                               idxn.at[pl.ds(k * SAMP_W, SAMP_W)], psem))
  for h in hs:
    h.wait()

  gh = [None] * NCH

  def start_chunk(ch):
    b = ch & 1
    hh = [
        pltpu.async_copy(u_hbm.at[idxu.at[pl.ds(ch * C, C)]],
                         ubufs[b], gsem[b]),
        pltpu.async_copy(v_hbm.at[idxv.at[pl.ds(ch * C, C)]],
                         vbufs[b], gsem[b]),
    ]
    for k in range(NEG):
      hh.append(pltpu.async_copy(
          v_hbm.at[idxn.at[pl.ds(k * SAMP_W + ch * C, C)]],
          nbufs[b].at[pl.ds(k * C, C)], gsem[b]))
    gh[ch] = hh

  iota16 = lax.iota(jnp.int32, 16)

  def compute_chunk(ch):
    b = ch & 1
    ub, vb, nb = ubufs[b], vbufs[b], nbufs[b]

    def group_body(g, _):
      sbase = g * 16
      rowu = sbase + iota16
      rowk = [rowu + (k * C) for k in range(NEG)]
      tvec = tbuf[pl.ds(ch * C + sbase, 16)]
      z = jnp.zeros((16,), jnp.float32)

      def d_body(d, carry):
        accp, a0, a1, a2, a3, a4 = carry
        col = jnp.broadcast_to(d, (16,))
        fd = fsplat[d]
        uu = plsc.load_gather(ub, [rowu, col])
        eu = uu + _sin16(tvec * fd)
        accp = accp + eu * plsc.load_gather(vb, [rowu, col])
        a0 = a0 + eu * plsc.load_gather(nb, [rowk[0], col])
        a1 = a1 + eu * plsc.load_gather(nb, [rowk[1], col])
        a2 = a2 + eu * plsc.load_gather(nb, [rowk[2], col])
        a3 = a3 + eu * plsc.load_gather(nb, [rowk[3], col])
        a4 = a4 + eu * plsc.load_gather(nb, [rowk[4], col])
        return accp, a0, a1, a2, a3, a4

      accp, a0, a1, a2, a3, a4 = lax.fori_loop(
          0, D, d_body, (z, z, z, z, z, z))
      pos_w[pl.ds(ch * C + sbase, 16)] = accp
      for k, acc in enumerate((a0, a1, a2, a3, a4)):
        neg_w[pl.ds(k * SAMP_W + ch * C + sbase, 16)] = acc
      return 0

    lax.fori_loop(0, GROUPS, group_body, 0)

  start_chunk(0)
  for ch in range(NCH):
    if ch + 1 < NCH:
      start_chunk(ch + 1)
    for h in gh[ch]:
      h.wait()
    compute_chunk(ch)

  ws = [pltpu.async_copy(pos_w, ps_hbm.at[pl.ds(base, SAMP_W)], wsem)]
  for k in range(NEG):
    ws.append(pltpu.async_copy(neg_w.at[pl.ds(k * SAMP_W, SAMP_W)],
                               ns_hbm.at[pl.ds(k * B + base, SAMP_W)], wsem))
  for h in ws:
    h.wait()


def _sc_scores(u_table, v_table, pos_u, pos_v, neg_flat, time, freq_sp):
  mesh = plsc.VectorSubcoreMesh(core_axis_name="c", subcore_axis_name="s")
  out_type = [
      jax.ShapeDtypeStruct((B,), jnp.float32),
      jax.ShapeDtypeStruct((B * NEG,), jnp.float32),
  ]
  k = pl.kernel(
      _sc_body,
      out_type=out_type,
      mesh=mesh,
      compiler_params=pltpu.CompilerParams(needs_layout_passes=False),
      scratch_types=[
          pltpu.VMEM((SAMP_W,), jnp.int32),
          pltpu.VMEM((SAMP_W,), jnp.int32),
          pltpu.VMEM((SAMP_W * NEG,), jnp.int32),
          pltpu.VMEM((SAMP_W,), jnp.float32),
          pltpu.VMEM((D, 16), jnp.float32),
          pltpu.VMEM((SAMP_W,), jnp.float32),
          pltpu.VMEM((SAMP_W * NEG,), jnp.float32),
          pltpu.VMEM((C, D), jnp.float32),
          pltpu.VMEM((C, D), jnp.float32),
          pltpu.VMEM((C * NEG, D), jnp.float32),
          pltpu.VMEM((C, D), jnp.float32),
          pltpu.VMEM((C, D), jnp.float32),
          pltpu.VMEM((C * NEG, D), jnp.float32),
          pltpu.SemaphoreType.DMA,
          pltpu.SemaphoreType.DMA,
          pltpu.SemaphoreType.DMA,
          pltpu.SemaphoreType.DMA,
      ],
  )
  return k(u_table, v_table, pos_u, pos_v, neg_flat, time, freq_sp)


def _tc_loss_body(p_ref, n_ref, o_ref):
  ps = jnp.clip(p_ref[...], -10.0, 10.0)
  acc = jnp.sum(jnp.log1p(jnp.exp(-ps)))
  ns = jnp.clip(n_ref[...], -10.0, 10.0)
  acc = acc + jnp.sum(jnp.log1p(jnp.exp(ns)))
  o_ref[0, 0] = acc


def _tc_loss(pos_s, neg_s):
  return pl.pallas_call(
      _tc_loss_body,
      in_specs=[
          pl.BlockSpec((B // D, D), lambda: (0, 0)),
          pl.BlockSpec((B * NEG // D, D), lambda: (0, 0)),
      ],
      out_specs=pl.BlockSpec((1, 1), lambda: (0, 0),
                             memory_space=pltpu.SMEM),
      out_shape=jax.ShapeDtypeStruct((1, 1), jnp.float32),
  )(pos_s.reshape(B // D, D), neg_s.reshape(B * NEG // D, D))


def kernel(u_table, v_table, freq_emb, time, pos_u, pos_v, neg_v):
  pu = pos_u.astype(jnp.int32)
  pv = pos_v.astype(jnp.int32)
  nf = neg_v.T.reshape(-1).astype(jnp.int32)
  fsp = jnp.broadcast_to(freq_emb[:, None], (D, 16))
  pos_s, neg_s = _sc_scores(u_table, v_table, pu, pv, nf, time, fsp)
  acc = _tc_loss(pos_s, neg_s)
  return acc[0, 0] / B


# speedup vs baseline: 2.3734x; 2.3734x over previous
import functools

import jax
import jax.numpy as jnp
from jax import lax
from jax.experimental import pallas as pl
from jax.experimental.pallas import tpu as pltpu
from jax.experimental.pallas import tpu_sc as plsc

VOCAB = 100000
D = 128
B = 16384
NEG = 5

NC = 2
NS = 16
NW = NC * NS
CHUNK = 128

U_PER_W = B // NW
N_PER_W = B * NEG // NW


DEPTH = 6
N_CHUNKS = (2 * U_PER_W + N_PER_W) // CHUNK


def _sc_gather_body(u_hbm, v_hbm, pu_hbm, pv_hbm, nf_hbm,
                    ug_hbm, vg_hbm, ng_hbm,
                    idxu, idxv, idxn,
                    b0, b1, b2, b3, b4, b5,
                    g0, g1, g2, g3, g4, g5,
                    w0, w1, w2, w3, w4, w5):
  bufs = [b0, b1, b2, b3, b4, b5]
  gsem = [g0, g1, g2, g3, g4, g5]
  wsem = [w0, w1, w2, w3, w4, w5]
  c = lax.axis_index("c")
  s = lax.axis_index("s")
  wid = s * NC + c

  h0 = pltpu.async_copy(pu_hbm.at[pl.ds(wid * U_PER_W, U_PER_W)], idxu, wsem[0])
  h1 = pltpu.async_copy(pv_hbm.at[pl.ds(wid * U_PER_W, U_PER_W)], idxv, wsem[1])
  h2 = pltpu.async_copy(nf_hbm.at[pl.ds(wid * N_PER_W, N_PER_W)], idxn, wsem[2])
  h0.wait()
  h1.wait()
  h2.wait()

  chunks = []
  for j in range(U_PER_W // CHUNK):
    chunks.append((u_hbm, idxu, j * CHUNK, ug_hbm, wid * U_PER_W + j * CHUNK))
  for j in range(U_PER_W // CHUNK):
    chunks.append((v_hbm, idxv, j * CHUNK, vg_hbm, wid * U_PER_W + j * CHUNK))
  for j in range(N_PER_W // CHUNK):
    chunks.append((v_hbm, idxn, j * CHUNK, ng_hbm, wid * N_PER_W + j * CHUNK))

  gh = [None] * N_CHUNKS
  wh = [None] * N_CHUNKS

  def start_gather(t):
    tbl, iref, ioff, _, _ = chunks[t]
    b = t % DEPTH
    gh[t] = pltpu.async_copy(tbl.at[iref.at[pl.ds(ioff, CHUNK)]],
                             bufs[b], gsem[b])

  for t in range(DEPTH):
    start_gather(t)
  for t in range(N_CHUNKS):
    b = t % DEPTH
    gh[t].wait()
    _, _, _, out_hbm, ooff = chunks[t]
    wh[t] = pltpu.async_copy(bufs[b], out_hbm.at[pl.ds(ooff, CHUNK)], wsem[b])
    if t + DEPTH < N_CHUNKS:
      wh[t].wait()
      start_gather(t + DEPTH)
  for t in range(N_CHUNKS - DEPTH, N_CHUNKS):
    wh[t].wait()


def _sc_gather(u_table, v_table, pos_u, pos_v, neg_flat):
  mesh = plsc.VectorSubcoreMesh(core_axis_name="c", subcore_axis_name="s")
  out_type = [
      jax.ShapeDtypeStruct((B, D), jnp.float32),
      jax.ShapeDtypeStruct((B, D), jnp.float32),
      jax.ShapeDtypeStruct((B * NEG, D), jnp.float32),
  ]
  k = pl.kernel(
      _sc_gather_body,
      out_type=out_type,
      mesh=mesh,
      scratch_types=(
          [pltpu.VMEM((U_PER_W,), jnp.int32),
           pltpu.VMEM((U_PER_W,), jnp.int32),
           pltpu.VMEM((N_PER_W,), jnp.int32)]
          + [pltpu.VMEM((CHUNK, D), jnp.float32) for _ in range(DEPTH)]
          + [pltpu.SemaphoreType.DMA for _ in range(2 * DEPTH)]
      ),
  )
  return k(u_table, v_table, pos_u, pos_v, neg_flat)


CB = 512
NBLK = B // CB

_TWO_PI = 6.283185307179586
_PI = 3.141592653589793
_HALF_PI = 1.5707963267948966
_S3 = -0.16666666666666666
_S5 = 0.008333333333333333
_S7 = -0.0001984126984126984
_S9 = 2.7557319223985893e-06


def _sin_poly(x):
  r = lax.rem(x, jnp.float32(_TWO_PI))
  r = jnp.where(r > _PI, r - _TWO_PI, r)
  r = jnp.where(r < -_PI, r + _TWO_PI, r)
  r = jnp.where(r > _HALF_PI, _PI - r, r)
  r = jnp.where(r < -_HALF_PI, -_PI - r, r)
  r2 = r * r
  p = jnp.float32(_S9)
  p = p * r2 + _S7
  p = p * r2 + _S5
  p = p * r2 + _S3
  p = p * r2 + 1.0
  return p * r


def _tc_loss_body(t_ref, f_ref, ug_ref, vg_ref, n0, n1, n2, n3, n4, o_ref):
  i = pl.program_id(0)
  te = _sin_poly(t_ref[...] * f_ref[...])
  eu = ug_ref[...] + te
  s = jnp.clip(jnp.sum(eu * vg_ref[...], axis=-1), -10.0, 10.0)
  acc = jnp.sum(jnp.log1p(jnp.exp(-s)))
  for nref in (n0, n1, n2, n3, n4):
    ns = jnp.clip(jnp.sum(nref[...] * eu, axis=-1), -10.0, 10.0)
    acc = acc + jnp.sum(jnp.log1p(jnp.exp(ns)))

  @pl.when(i == 0)
  def _():
    o_ref[0, 0] = 0.0

  o_ref[0, 0] += acc


def _tc_loss(time, freq_emb, ug, vg, ng):
  t2 = time.reshape(B, 1)
  f2 = freq_emb.reshape(1, D)
  in_specs = [
      pl.BlockSpec((CB, 1), lambda i: (i, 0)),
      pl.BlockSpec((1, D), lambda i: (0, 0)),
      pl.BlockSpec((CB, D), lambda i: (i, 0)),
      pl.BlockSpec((CB, D), lambda i: (i, 0)),
  ] + [
      pl.BlockSpec((CB, D), lambda i, k=k: (k * NBLK + i, 0))
      for k in range(NEG)
  ]
  out = pl.pallas_call(
      _tc_loss_body,
      grid=(NBLK,),
      in_specs=in_specs,
      out_specs=pl.BlockSpec((1, 1), lambda i: (0, 0),
                             memory_space=pltpu.SMEM),
      out_shape=jax.ShapeDtypeStruct((1, 1), jnp.float32),
  )(t2, f2, ug, vg, ng, ng, ng, ng, ng)
  return out


def kernel(u_table, v_table, freq_emb, time, pos_u, pos_v, neg_v):
  pu = pos_u.astype(jnp.int32)
  pv = pos_v.astype(jnp.int32)
  nf = neg_v.T.reshape(-1).astype(jnp.int32)
  ug, vg, ng = _sc_gather(u_table, v_table, pu, pv, nf)
  acc = _tc_loss(time, freq_emb, ug, vg, ng)
  return acc[0, 0] / B


# speedup vs baseline: 2.4966x; 1.0519x over previous
import jax
import jax.numpy as jnp
from jax import lax
from jax.experimental import pallas as pl
from jax.experimental.pallas import tpu as pltpu
from jax.experimental.pallas import tpu_sc as plsc

VOCAB = 100000
D = 128
B = 16384
NEG = 5
HALVES = 2
BH = B // HALVES

NC = 2
NS = 16
NW = NC * NS
CHUNK = 128
DEPTH = 6

U_PER_W = BH // NW
N_PER_W = BH * NEG // NW
N_CHUNKS = (2 * U_PER_W + N_PER_W) // CHUNK


def _sc_gather_body(u_hbm, v_hbm, pu_hbm, pv_hbm, nf_hbm,
                    ug_hbm, vg_hbm, ng_hbm,
                    idxu, idxv, idxn,
                    b0, b1, b2, b3, b4, b5,
                    g0, g1, g2, g3, g4, g5,
                    w0, w1, w2, w3, w4, w5):
  bufs = [b0, b1, b2, b3, b4, b5]
  gsem = [g0, g1, g2, g3, g4, g5]
  wsem = [w0, w1, w2, w3, w4, w5]
  c = lax.axis_index("c")
  s = lax.axis_index("s")
  wid = s * NC + c

  h0 = pltpu.async_copy(pu_hbm.at[pl.ds(wid * U_PER_W, U_PER_W)], idxu, wsem[0])
  h1 = pltpu.async_copy(pv_hbm.at[pl.ds(wid * U_PER_W, U_PER_W)], idxv, wsem[1])
  h2 = pltpu.async_copy(nf_hbm.at[pl.ds(wid * N_PER_W, N_PER_W)], idxn, wsem[2])
  h0.wait()
  h1.wait()
  h2.wait()

  chunks = []
  for j in range(U_PER_W // CHUNK):
    chunks.append((u_hbm, idxu, j * CHUNK, ug_hbm, wid * U_PER_W + j * CHUNK))
  for j in range(U_PER_W // CHUNK):
    chunks.append((v_hbm, idxv, j * CHUNK, vg_hbm, wid * U_PER_W + j * CHUNK))
  for j in range(N_PER_W // CHUNK):
    chunks.append((v_hbm, idxn, j * CHUNK, ng_hbm, wid * N_PER_W + j * CHUNK))

  gh = [None] * N_CHUNKS
  wh = [None] * N_CHUNKS

  def start_gather(t):
    tbl, iref, ioff, _, _ = chunks[t]
    b = t % DEPTH
    gh[t] = pltpu.async_copy(tbl.at[iref.at[pl.ds(ioff, CHUNK)]],
                             bufs[b], gsem[b])

  for t in range(DEPTH):
    start_gather(t)
  for t in range(N_CHUNKS):
    b = t % DEPTH
    gh[t].wait()
    _, _, _, out_hbm, ooff = chunks[t]
    wh[t] = pltpu.async_copy(bufs[b], out_hbm.at[pl.ds(ooff, CHUNK)], wsem[b])
    if t + DEPTH < N_CHUNKS:
      wh[t].wait()
      start_gather(t + DEPTH)
  for t in range(N_CHUNKS - DEPTH, N_CHUNKS):
    wh[t].wait()


def _sc_gather(u_table, v_table, pos_u, pos_v, neg_flat):
  mesh = plsc.VectorSubcoreMesh(core_axis_name="c", subcore_axis_name="s")
  out_type = [
      jax.ShapeDtypeStruct((BH, D), jnp.float32),
      jax.ShapeDtypeStruct((BH, D), jnp.float32),
      jax.ShapeDtypeStruct((BH * NEG, D), jnp.float32),
  ]
  k = pl.kernel(
      _sc_gather_body,
      out_type=out_type,
      mesh=mesh,
      scratch_types=(
          [pltpu.VMEM((U_PER_W,), jnp.int32),
           pltpu.VMEM((U_PER_W,), jnp.int32),
           pltpu.VMEM((N_PER_W,), jnp.int32)]
          + [pltpu.VMEM((CHUNK, D), jnp.float32) for _ in range(DEPTH)]
          + [pltpu.SemaphoreType.DMA for _ in range(2 * DEPTH)]
      ),
  )
  return k(u_table, v_table, pos_u, pos_v, neg_flat)


CB = 512
NBLK = BH // CB

_TWO_PI = 6.283185307179586
_PI = 3.141592653589793
_HALF_PI = 1.5707963267948966
_S3 = -0.16666666666666666
_S5 = 0.008333333333333333
_S7 = -0.0001984126984126984
_S9 = 2.7557319223985893e-06


def _sin_poly(x):
  r = lax.rem(x, jnp.float32(_TWO_PI))
  r = jnp.where(r > _PI, r - _TWO_PI, r)
  r = jnp.where(r < -_PI, r + _TWO_PI, r)
  r = jnp.where(r > _HALF_PI, _PI - r, r)
  r = jnp.where(r < -_HALF_PI, -_PI - r, r)
  r2 = r * r
  p = jnp.float32(_S9)
  p = p * r2 + _S7
  p = p * r2 + _S5
  p = p * r2 + _S3
  p = p * r2 + 1.0
  return p * r


def _tc_loss_body(t_ref, f_ref, ug_ref, vg_ref, n0, n1, n2, n3, n4, o_ref):
  i = pl.program_id(0)
  te = _sin_poly(t_ref[...] * f_ref[...])
  eu = ug_ref[...] + te
  s = jnp.clip(jnp.sum(eu * vg_ref[...], axis=-1), -10.0, 10.0)
  acc = jnp.sum(jnp.log1p(jnp.exp(-s)))
  for nref in (n0, n1, n2, n3, n4):
    ns = jnp.clip(jnp.sum(nref[...] * eu, axis=-1), -10.0, 10.0)
    acc = acc + jnp.sum(jnp.log1p(jnp.exp(ns)))

  @pl.when(i == 0)
  def _():
    o_ref[0, 0] = 0.0

  o_ref[0, 0] += acc


def _tc_loss(time_h, freq_emb, ug, vg, ng):
  t2 = time_h.reshape(BH, 1)
  f2 = freq_emb.reshape(1, D)
  in_specs = [
      pl.BlockSpec((CB, 1), lambda i: (i, 0)),
      pl.BlockSpec((1, D), lambda i: (0, 0)),
      pl.BlockSpec((CB, D), lambda i: (i, 0)),
      pl.BlockSpec((CB, D), lambda i: (i, 0)),
  ] + [
      pl.BlockSpec((CB, D), lambda i, k=k: (k * NBLK + i, 0))
      for k in range(NEG)
  ]
  out = pl.pallas_call(
      _tc_loss_body,
      grid=(NBLK,),
      in_specs=in_specs,
      out_specs=pl.BlockSpec((1, 1), lambda i: (0, 0),
                             memory_space=pltpu.SMEM),
      out_shape=jax.ShapeDtypeStruct((1, 1), jnp.float32),
  )(t2, f2, ug, vg, ng, ng, ng, ng, ng)
  return out


def kernel(u_table, v_table, freq_emb, time, pos_u, pos_v, neg_v):
  pu = pos_u.astype(jnp.int32)
  pv = pos_v.astype(jnp.int32)
  nvi = neg_v.astype(jnp.int32)

  gathered = []
  for h in range(HALVES):
    lo = h * BH
    nf = nvi[lo:lo + BH].T.reshape(-1)
    gathered.append(
        _sc_gather(u_table, v_table, pu[lo:lo + BH], pv[lo:lo + BH], nf))

  acc = jnp.float32(0.0)
  for h in range(HALVES):
    ug, vg, ng = gathered[h]
    acc = acc + _tc_loss(time[h * BH:(h + 1) * BH], freq_emb, ug, vg, ng)[0, 0]
  return acc / B


# speedup vs baseline: 2.6187x; 1.0489x over previous
import jax
import jax.numpy as jnp
from jax import lax
from jax.experimental import pallas as pl
from jax.experimental.pallas import tpu as pltpu
from jax.experimental.pallas import tpu_sc as plsc

VOCAB = 100000
D = 128
B = 16384
NEG = 5
HALVES = 2
BH = B // HALVES

NC = 2
NS = 16
NW = NC * NS
CHUNK = 128
DEPTH = 6

U_PER_W = BH // NW
N_PER_W = BH * NEG // NW
N_CHUNKS = (2 * U_PER_W + N_PER_W) // CHUNK


def _sc_gather_body(u_hbm, v_hbm, pu_hbm, pv_hbm, nf_hbm,
                    ug_hbm, vg_hbm, ng_hbm,
                    idxu, idxv, idxn,
                    b0, b1, b2, b3, b4, b5,
                    g0, g1, g2, g3, g4, g5,
                    w0, w1, w2, w3, w4, w5):
  bufs = [b0, b1, b2, b3, b4, b5]
  gsem = [g0, g1, g2, g3, g4, g5]
  wsem = [w0, w1, w2, w3, w4, w5]
  c = lax.axis_index("c")
  s = lax.axis_index("s")
  wid = s * NC + c

  h0 = pltpu.async_copy(pu_hbm.at[pl.ds(wid * U_PER_W, U_PER_W)], idxu, wsem[0])
  h1 = pltpu.async_copy(pv_hbm.at[pl.ds(wid * U_PER_W, U_PER_W)], idxv, wsem[1])
  h2 = pltpu.async_copy(nf_hbm.at[pl.ds(wid * N_PER_W, N_PER_W)], idxn, wsem[2])
  h0.wait()
  h1.wait()
  h2.wait()

  chunks = []
  for j in range(U_PER_W // CHUNK):
    chunks.append((u_hbm, idxu, j * CHUNK, ug_hbm, wid * U_PER_W + j * CHUNK))
  for j in range(U_PER_W // CHUNK):
    chunks.append((v_hbm, idxv, j * CHUNK, vg_hbm, wid * U_PER_W + j * CHUNK))
  for j in range(N_PER_W // CHUNK):
    chunks.append((v_hbm, idxn, j * CHUNK, ng_hbm, wid * N_PER_W + j * CHUNK))

  gh = [None] * N_CHUNKS
  wh = [None] * N_CHUNKS

  def start_gather(t):
    tbl, iref, ioff, _, _ = chunks[t]
    b = t % DEPTH
    gh[t] = pltpu.async_copy(tbl.at[iref.at[pl.ds(ioff, CHUNK)]],
                             bufs[b], gsem[b])

  for t in range(DEPTH):
    start_gather(t)
  for t in range(N_CHUNKS):
    b = t % DEPTH
    gh[t].wait()
    _, _, _, out_hbm, ooff = chunks[t]
    wh[t] = pltpu.async_copy(bufs[b], out_hbm.at[pl.ds(ooff, CHUNK)], wsem[b])
    if t + DEPTH < N_CHUNKS:
      wh[t].wait()
      start_gather(t + DEPTH)
  for t in range(N_CHUNKS - DEPTH, N_CHUNKS):
    wh[t].wait()


def _sc_gather(u_table, v_table, pos_u, pos_v, neg_flat):
  mesh = plsc.VectorSubcoreMesh(core_axis_name="c", subcore_axis_name="s")
  out_type = [
      jax.ShapeDtypeStruct((BH, D), jnp.float32),
      jax.ShapeDtypeStruct((BH, D), jnp.float32),
      jax.ShapeDtypeStruct((BH * NEG, D), jnp.float32),
  ]
  k = pl.kernel(
      _sc_gather_body,
      out_type=out_type,
      mesh=mesh,
      scratch_types=(
          [pltpu.VMEM((U_PER_W,), jnp.int32),
           pltpu.VMEM((U_PER_W,), jnp.int32),
           pltpu.VMEM((N_PER_W,), jnp.int32)]
          + [pltpu.VMEM((CHUNK, D), jnp.float32) for _ in range(DEPTH)]
          + [pltpu.SemaphoreType.DMA for _ in range(2 * DEPTH)]
      ),
  )
  return k(u_table, v_table, pos_u, pos_v, neg_flat)


CB = 512
NBLK = BH // CB

_TWO_PI = 6.283185307179586
_PI = 3.141592653589793
_HALF_PI = 1.5707963267948966
_S3 = -0.16666666666666666
_S5 = 0.008333333333333333
_S7 = -0.0001984126984126984
_S9 = 2.7557319223985893e-06


def _sin_poly(x):
  r = lax.rem(x, jnp.float32(_TWO_PI))
  r = jnp.where(r > _PI, r - _TWO_PI, r)
  r = jnp.where(r < -_PI, r + _TWO_PI, r)
  r = jnp.where(r > _HALF_PI, _PI - r, r)
  r = jnp.where(r < -_HALF_PI, -_PI - r, r)
  r2 = r * r
  p = jnp.float32(_S9)
  p = p * r2 + _S7
  p = p * r2 + _S5
  p = p * r2 + _S3
  p = p * r2 + 1.0
  return p * r


def _tc_loss_body(t_ref, f_ref, ug_ref, vg_ref, n0, n1, n2, n3, n4, o_ref):
  i = pl.program_id(0)
  te = _sin_poly(t_ref[...] * f_ref[...])
  eu = ug_ref[...] + te
  ones = jnp.ones((D, 1), jnp.float32)
  s = jnp.clip(jnp.dot(eu * vg_ref[...], ones,
                       preferred_element_type=jnp.float32), -10.0, 10.0)
  acc = jnp.sum(jnp.log1p(jnp.exp(-s)))
  for nref in (n0, n1, n2, n3, n4):
    ns = jnp.clip(jnp.dot(nref[...] * eu, ones,
                          preferred_element_type=jnp.float32), -10.0, 10.0)
    acc = acc + jnp.sum(jnp.log1p(jnp.exp(ns)))

  @pl.when(i == 0)
  def _():
    o_ref[0, 0] = 0.0

  o_ref[0, 0] += acc


def _tc_loss(time_h, freq_emb, ug, vg, ng):
  t2 = time_h.reshape(BH, 1)
  f2 = freq_emb.reshape(1, D)
  in_specs = [
      pl.BlockSpec((CB, 1), lambda i: (i, 0)),
      pl.BlockSpec((1, D), lambda i: (0, 0)),
      pl.BlockSpec((CB, D), lambda i: (i, 0)),
      pl.BlockSpec((CB, D), lambda i: (i, 0)),
  ] + [
      pl.BlockSpec((CB, D), lambda i, k=k: (k * NBLK + i, 0))
      for k in range(NEG)
  ]
  out = pl.pallas_call(
      _tc_loss_body,
      grid=(NBLK,),
      in_specs=in_specs,
      out_specs=pl.BlockSpec((1, 1), lambda i: (0, 0),
                             memory_space=pltpu.SMEM),
      out_shape=jax.ShapeDtypeStruct((1, 1), jnp.float32),
  )(t2, f2, ug, vg, ng, ng, ng, ng, ng)
  return out


def kernel(u_table, v_table, freq_emb, time, pos_u, pos_v, neg_v):
  pu = pos_u.astype(jnp.int32)
  pv = pos_v.astype(jnp.int32)
  nvi = neg_v.astype(jnp.int32)

  gathered = []
  for h in range(HALVES):
    lo = h * BH
    nf = nvi[lo:lo + BH].T.reshape(-1)
    gathered.append(
        _sc_gather(u_table, v_table, pu[lo:lo + BH], pv[lo:lo + BH], nf))

  acc = jnp.float32(0.0)
  for h in range(HALVES):
    ug, vg, ng = gathered[h]
    acc = acc + _tc_loss(time[h * BH:(h + 1) * BH], freq_emb, ug, vg, ng)[0, 0]
  return acc / B
